# baseline (speedup 1.0000x reference)
import functools

import jax
import jax.numpy as jnp
from jax import lax
from jax.experimental import pallas as pl
from jax.experimental.pallas import tpu as pltpu
from jax.experimental.pallas import tpu_sc as plsc

N_NODES = 10000
N_EDGES = 160000
D = 256
H = 128

NC = 2
NS = 16
EB = 128
TILE_BLOCKS = 80
GRP = 5
N_GRPS = TILE_BLOCKS // GRP
NSL = 3
TOTAL_BLOCKS = TILE_BLOCKS * NS
E_PAD = TOTAL_BLOCKS * EB
ACC_ROWS = 10016
STRIPE = 632
LAST_INIT = ACC_ROWS - (NS - 1) * STRIPE
LAST_OUT = N_NODES - (NS - 1) * STRIPE
DUMMY_ROW = N_NODES


def _mm_body(x_ref, w_ref, o_ref):
    o_ref[...] = jnp.dot(x_ref[...], w_ref[...],
                         preferred_element_type=jnp.float32)


def _matmul_halves(inputs, weight):
    return pl.pallas_call(
        _mm_body,
        grid=(2, 10),
        in_specs=[
            pl.BlockSpec((1000, D), lambda h, i: (i, 0)),
            pl.BlockSpec((D, H), lambda h, i: (0, h)),
        ],
        out_specs=pl.BlockSpec((1000, H), lambda h, i: (h * 10 + i, 0)),
        out_shape=jax.ShapeDtypeStruct((2 * N_NODES, H), jnp.float32),
    )(inputs, weight)


def _sc_body(t2_hbm, idx_hbm, bias_hbm, out_hbm,
             acc, idxv, rbuf, gsem, ssem):
    c = lax.axis_index("c")
    s = lax.axis_index("s")

    base = s * STRIPE

    @pl.when(s < NS - 1)
    def _():
        pltpu.sync_copy(bias_hbm.at[c], acc.at[pl.ds(base, STRIPE)])

    @pl.when(s == NS - 1)
    def _():
        pltpu.sync_copy(bias_hbm.at[c, pl.ds(0, LAST_INIT)],
                        acc.at[pl.ds(base, LAST_INIT)])

    plsc.subcore_barrier()

    tb = s * TILE_BLOCKS

    def g_start(b, sl):
        pltpu.async_copy(t2_hbm.at[idxv.at[b, 0]],
                         rbuf.at[pl.ds(sl * EB, EB)], gsem)

    def g_wait():
        pltpu.make_async_copy(t2_hbm.at[pl.ds(0, EB)],
                              rbuf.at[pl.ds(0, EB)], gsem).wait()

    def s_start(b, sl):
        pltpu.async_copy(rbuf.at[pl.ds(sl * EB, EB)],
                         acc.at[idxv.at[b, 1]], ssem, add=True)

    def s_wait():
        pltpu.make_async_copy(rbuf.at[pl.ds(0, EB)],
                              acc.at[pl.ds(0, EB)], ssem).wait()

    def grp(gi, carry):
        pltpu.sync_copy(idx_hbm.at[c, pl.ds(tb + gi * GRP, GRP)], idxv)
        g_start(0, 0)
        g_start(1, 1)
        for b in range(GRP):
            sl = b % NSL
            g_wait()
            s_start(b, sl)
            if b + 2 < GRP:
                if b >= 1:
                    s_wait()
                g_start(b + 2, (b + 2) % NSL)
        s_wait()
        s_wait()
        s_wait()
        return carry

    lax.fori_loop(0, N_GRPS, grp, None)

    plsc.subcore_barrier()

    @pl.when(s < NS - 1)
    def _():
        pltpu.sync_copy(acc.at[pl.ds(base, STRIPE)],
                        out_hbm.at[pl.ds(base, STRIPE), pl.ds(c * H, H)])

    @pl.when(s == NS - 1)
    def _():
        pltpu.sync_copy(acc.at[pl.ds(base, LAST_OUT)],
                        out_hbm.at[pl.ds(base, LAST_OUT), pl.ds(c * H, H)])


_sc_scatter = functools.partial(
    pl.kernel,
    out_type=jax.ShapeDtypeStruct((N_NODES, D), jnp.float32),
    mesh=plsc.VectorSubcoreMesh(core_axis_name="c", subcore_axis_name="s",
                                num_cores=NC, num_subcores=NS),
    scratch_types=[
        pltpu.VMEM_SHARED((ACC_ROWS, H), jnp.float32),
        pltpu.VMEM((GRP, 2, EB), jnp.int32),
        pltpu.VMEM((NSL * EB, H), jnp.float32),
        pltpu.SemaphoreType.DMA,
        pltpu.SemaphoreType.DMA,
    ],
)(_sc_body)


def kernel(inputs, edge_index, weight, bias):
    row = edge_index[0].astype(jnp.int32)
    col = edge_index[1].astype(jnp.int32)

    rowp = jnp.full((E_PAD,), DUMMY_ROW, jnp.int32).at[:N_EDGES].set(row)
    colp0 = jnp.zeros((E_PAD,), jnp.int32).at[:N_EDGES].set(col)
    col2 = jnp.stack([colp0, colp0 + N_NODES]).reshape(NC, TOTAL_BLOCKS, EB)
    row2 = jnp.broadcast_to(rowp.reshape(TOTAL_BLOCKS, EB),
                            (NC, TOTAL_BLOCKS, EB))
    idx = jnp.stack([col2, row2], axis=2)

    t2 = _matmul_halves(inputs, weight)
    biasfill = jnp.broadcast_to(bias.reshape(NC, 1, H), (NC, STRIPE, H))
    return _sc_scatter(t2, idx, biasfill)


# speedup vs baseline: 1.0003x; 1.0003x over previous
import functools

import jax
import jax.numpy as jnp
from jax import lax
from jax.experimental import pallas as pl
from jax.experimental.pallas import tpu as pltpu
from jax.experimental.pallas import tpu_sc as plsc

N_NODES = 10000
N_EDGES = 160000
D = 256
H = 128

NC = 2
NS = 16
EB = 128
TILE_BLOCKS = 80
GRP = 5
N_GRPS = TILE_BLOCKS // GRP
NSL = 3
TOTAL_BLOCKS = TILE_BLOCKS * NS
E_PAD = TOTAL_BLOCKS * EB
ACC_ROWS = 10016
STRIPE = 632
LAST_INIT = ACC_ROWS - (NS - 1) * STRIPE
LAST_OUT = N_NODES - (NS - 1) * STRIPE
DUMMY_ROW = N_NODES


def _mm_body(x_ref, w_ref, o_ref):
    o_ref[...] = jnp.dot(x_ref[...], w_ref[...],
                         preferred_element_type=jnp.float32)


def _matmul_halves(inputs, weight):
    return pl.pallas_call(
        _mm_body,
        grid=(2, 10),
        in_specs=[
            pl.BlockSpec((1000, D), lambda h, i: (i, 0)),
            pl.BlockSpec((D, H), lambda h, i: (0, h)),
        ],
        out_specs=pl.BlockSpec((1000, H), lambda h, i: (h * 10 + i, 0)),
        out_shape=jax.ShapeDtypeStruct((2 * N_NODES, H), jnp.float32),
    )(inputs, weight)


def _sc_body(t2_hbm, idx_hbm, bias_hbm, out_hbm,
             acc, idxv, rbuf, gsem, ssem):
    c = lax.axis_index("c")
    s = lax.axis_index("s")

    base = s * STRIPE

    @pl.when(s < NS - 1)
    def _():
        pltpu.sync_copy(bias_hbm.at[c], acc.at[pl.ds(base, STRIPE)])

    @pl.when(s == NS - 1)
    def _():
        pltpu.sync_copy(bias_hbm.at[c, pl.ds(0, LAST_INIT)],
                        acc.at[pl.ds(base, LAST_INIT)])

    plsc.subcore_barrier()

    tb = s * TILE_BLOCKS

    def g_start(b, sl):
        pltpu.async_copy(t2_hbm.at[idxv.at[b, 0]],
                         rbuf.at[pl.ds(sl * EB, EB)], gsem)

    def g_wait():
        pltpu.make_async_copy(t2_hbm.at[pl.ds(0, EB)],
                              rbuf.at[pl.ds(0, EB)], gsem).wait()

    def s_start(b, sl):
        pltpu.async_copy(rbuf.at[pl.ds(sl * EB, EB)],
                         acc.at[idxv.at[b, 1]], ssem, add=True)

    def s_wait():
        pltpu.make_async_copy(rbuf.at[pl.ds(0, EB)],
                              acc.at[pl.ds(0, EB)], ssem).wait()

    coff = c * N_NODES

    def grp(gi, carry):
        pltpu.sync_copy(idx_hbm.at[pl.ds(tb + gi * GRP, GRP)], idxv)
        for b in range(GRP):
            for k in range(EB // 16):
                idxv[b, 0, pl.ds(k * 16, 16)] = (
                    idxv[b, 0, pl.ds(k * 16, 16)] + coff)
        g_start(0, 0)
        g_start(1, 1)
        for b in range(GRP):
            sl = b % NSL
            g_wait()
            s_start(b, sl)
            if b + 2 < GRP:
                if b >= 1:
                    s_wait()
                g_start(b + 2, (b + 2) % NSL)
        s_wait()
        s_wait()
        s_wait()
        return carry

    lax.fori_loop(0, N_GRPS, grp, None)

    plsc.subcore_barrier()

    @pl.when(s < NS - 1)
    def _():
        pltpu.sync_copy(acc.at[pl.ds(base, STRIPE)],
                        out_hbm.at[pl.ds(base, STRIPE), pl.ds(c * H, H)])

    @pl.when(s == NS - 1)
    def _():
        pltpu.sync_copy(acc.at[pl.ds(base, LAST_OUT)],
                        out_hbm.at[pl.ds(base, LAST_OUT), pl.ds(c * H, H)])


_sc_scatter = functools.partial(
    pl.kernel,
    out_type=jax.ShapeDtypeStruct((N_NODES, D), jnp.float32),
    mesh=plsc.VectorSubcoreMesh(core_axis_name="c", subcore_axis_name="s",
                                num_cores=NC, num_subcores=NS),
    scratch_types=[
        pltpu.VMEM_SHARED((ACC_ROWS, H), jnp.float32),
        pltpu.VMEM((GRP, 2, EB), jnp.int32),
        pltpu.VMEM((NSL * EB, H), jnp.float32),
        pltpu.SemaphoreType.DMA,
        pltpu.SemaphoreType.DMA,
    ],
)(_sc_body)


def kernel(inputs, edge_index, weight, bias):
    row = edge_index[0].astype(jnp.int32)
    col = edge_index[1].astype(jnp.int32)

    rowp = jnp.full((E_PAD,), DUMMY_ROW, jnp.int32).at[:N_EDGES].set(row)
    colp = jnp.zeros((E_PAD,), jnp.int32).at[:N_EDGES].set(col)
    idx = jnp.stack([colp.reshape(TOTAL_BLOCKS, EB),
                     rowp.reshape(TOTAL_BLOCKS, EB)], axis=1)

    t2 = _matmul_halves(inputs, weight)
    biasfill = jnp.broadcast_to(bias.reshape(NC, 1, H), (NC, STRIPE, H))
    return _sc_scatter(t2, idx, biasfill)


# speedup vs baseline: 1.0025x; 1.0022x over previous
import functools

import jax
import jax.numpy as jnp
from jax import lax
from jax.experimental import pallas as pl
from jax.experimental.pallas import tpu as pltpu
from jax.experimental.pallas import tpu_sc as plsc

N_NODES = 10000
N_EDGES = 160000
D = 256
H = 128

NC = 2
NS = 16
EB = 128
TILE_BLOCKS = 80
GRP = 5
N_GRPS = TILE_BLOCKS // GRP
NSL = 3
TOTAL_BLOCKS = TILE_BLOCKS * NS
E_PAD = TOTAL_BLOCKS * EB
ACC_ROWS = 10016
STRIPE = 632
LAST_INIT = ACC_ROWS - (NS - 1) * STRIPE
LAST_OUT = N_NODES - (NS - 1) * STRIPE
DUMMY_ROW = N_NODES


def _mm_body(x_ref, w_ref, o_ref):
    o_ref[...] = jnp.dot(x_ref[...].astype(jnp.bfloat16),
                         w_ref[...].astype(jnp.bfloat16),
                         preferred_element_type=jnp.float32)


def _matmul_halves(inputs, weight):
    return pl.pallas_call(
        _mm_body,
        grid=(2, 10),
        in_specs=[
            pl.BlockSpec((1000, D), lambda h, i: (i, 0)),
            pl.BlockSpec((D, H), lambda h, i: (0, h)),
        ],
        out_specs=pl.BlockSpec((1000, H), lambda h, i: (h * 10 + i, 0)),
        out_shape=jax.ShapeDtypeStruct((2 * N_NODES, H), jnp.float32),
    )(inputs, weight)


def _sc_body(t2_hbm, idx_hbm, bias_hbm, out_hbm,
             acc, idxv, rbuf, gsem, ssem):
    c = lax.axis_index("c")
    s = lax.axis_index("s")
    base = s * STRIPE
    tb = s * TILE_BLOCKS
    coff = c * N_NODES

    def g_start(b, sl):
        pltpu.async_copy(t2_hbm.at[idxv.at[b, 0]],
                         rbuf.at[pl.ds(sl * EB, EB)], gsem)

    def g_wait():
        pltpu.make_async_copy(t2_hbm.at[pl.ds(0, EB)],
                              rbuf.at[pl.ds(0, EB)], gsem).wait()

    def s_start(b, sl):
        pltpu.async_copy(rbuf.at[pl.ds(sl * EB, EB)],
                         acc.at[idxv.at[b, 1]], ssem, add=True)

    def s_wait():
        pltpu.make_async_copy(rbuf.at[pl.ds(0, EB)],
                              acc.at[pl.ds(0, EB)], ssem).wait()

    def grp_prologue(gi):
        pltpu.sync_copy(idx_hbm.at[pl.ds(tb + gi * GRP, GRP)], idxv)
        for b in range(GRP):
            for k in range(EB // 16):
                idxv[b, 0, pl.ds(k * 16, 16)] = (
                    idxv[b, 0, pl.ds(k * 16, 16)] + coff)
        g_start(0, 0)
        g_start(1, 1)

    def grp_main():
        for b in range(GRP):
            sl = b % NSL
            g_wait()
            s_start(b, sl)
            if b + 2 < GRP:
                if b >= 1:
                    s_wait()
                g_start(b + 2, (b + 2) % NSL)
        s_wait()
        s_wait()
        s_wait()

    def grp(gi, carry):
        grp_prologue(gi)
        grp_main()
        return carry

    grp_prologue(0)

    @pl.when(s < NS - 1)
    def _():
        pltpu.sync_copy(bias_hbm.at[c], acc.at[pl.ds(base, STRIPE)])

    @pl.when(s == NS - 1)
    def _():
        pltpu.sync_copy(bias_hbm.at[c, pl.ds(0, LAST_INIT)],
                        acc.at[pl.ds(base, LAST_INIT)])

    plsc.subcore_barrier()

    grp_main()
    lax.fori_loop(1, N_GRPS, grp, None)

    plsc.subcore_barrier()

    @pl.when(s < NS - 1)
    def _():
        pltpu.sync_copy(acc.at[pl.ds(base, STRIPE)],
                        out_hbm.at[pl.ds(base, STRIPE), pl.ds(c * H, H)])

    @pl.when(s == NS - 1)
    def _():
        pltpu.sync_copy(acc.at[pl.ds(base, LAST_OUT)],
                        out_hbm.at[pl.ds(base, LAST_OUT), pl.ds(c * H, H)])


_sc_scatter = functools.partial(
    pl.kernel,
    out_type=jax.ShapeDtypeStruct((N_NODES, D), jnp.float32),
    mesh=plsc.VectorSubcoreMesh(core_axis_name="c", subcore_axis_name="s",
                                num_cores=NC, num_subcores=NS),
    scratch_types=[
        pltpu.VMEM_SHARED((ACC_ROWS, H), jnp.float32),
        pltpu.VMEM((GRP, 2, EB), jnp.int32),
        pltpu.VMEM((NSL * EB, H), jnp.float32),
        pltpu.SemaphoreType.DMA,
        pltpu.SemaphoreType.DMA,
    ],
)(_sc_body)


def kernel(inputs, edge_index, weight, bias):
    row = edge_index[0].astype(jnp.int32)
    col = edge_index[1].astype(jnp.int32)

    rowp = jnp.full((E_PAD,), DUMMY_ROW, jnp.int32).at[:N_EDGES].set(row)
    colp = jnp.zeros((E_PAD,), jnp.int32).at[:N_EDGES].set(col)
    idx = jnp.stack([colp.reshape(TOTAL_BLOCKS, EB),
                     rowp.reshape(TOTAL_BLOCKS, EB)], axis=1)

    t2 = _matmul_halves(inputs, weight)
    biasfill = jnp.broadcast_to(bias.reshape(NC, 1, H), (NC, STRIPE, H))
    return _sc_scatter(t2, idx, biasfill)


# speedup vs baseline: 1.0193x; 1.0168x over previous
import functools

import jax
import jax.numpy as jnp
from jax import lax
from jax.experimental import pallas as pl
from jax.experimental.pallas import tpu as pltpu
from jax.experimental.pallas import tpu_sc as plsc

N_NODES = 10000
N_EDGES = 160000
D = 256
H = 128

NC = 2
NS = 16
EB = 128
TILE_BLOCKS = 80
GRP = 5
N_GRPS = TILE_BLOCKS // GRP
NSL = 3
TOTAL_BLOCKS = TILE_BLOCKS * NS
E_PAD = TOTAL_BLOCKS * EB
ACC_ROWS = 10016
STRIPE = 632
LAST_INIT = ACC_ROWS - (NS - 1) * STRIPE
LAST_OUT = N_NODES - (NS - 1) * STRIPE
DUMMY_ROW = N_NODES


def _mm_body(x_ref, w_ref, o_ref):
    o_ref[...] = jnp.dot(x_ref[...].astype(jnp.bfloat16),
                         w_ref[...].astype(jnp.bfloat16),
                         preferred_element_type=jnp.float32)


def _matmul_halves(inputs, weight):
    return pl.pallas_call(
        _mm_body,
        grid=(2, 10),
        in_specs=[
            pl.BlockSpec((1000, D), lambda h, i: (i, 0)),
            pl.BlockSpec((D, H), lambda h, i: (0, h)),
        ],
        out_specs=pl.BlockSpec((1000, H), lambda h, i: (h * 10 + i, 0)),
        out_shape=jax.ShapeDtypeStruct((2 * N_NODES, H), jnp.float32),
    )(inputs, weight)


def _sc_body(t2_hbm, idx_hbm, bias_hbm, out_hbm,
             acc, idxv, rpriv, rbuf, gsem, ssem):
    c = lax.axis_index("c")
    s = lax.axis_index("s")
    base = s * STRIPE
    tb = s * TILE_BLOCKS
    coff = c * N_NODES

    def g_start(b, sl):
        pltpu.async_copy(t2_hbm.at[idxv.at[b, 0]],
                         rbuf.at[pl.ds(sl * EB, EB)], gsem)

    def g_wait():
        pltpu.make_async_copy(t2_hbm.at[pl.ds(0, EB)],
                              rbuf.at[pl.ds(0, EB)], gsem).wait()

    def s_start(b, sl):
        for k in range(EB // 16):
            rpriv[sl, pl.ds(k * 16, 16)] = idxv[b, 1, pl.ds(k * 16, 16)]
        pltpu.async_copy(rbuf.at[pl.ds(sl * EB, EB)],
                         acc.at[rpriv.at[sl]], ssem, add=True)

    def s_wait():
        pltpu.make_async_copy(rbuf.at[pl.ds(0, EB)],
                              acc.at[pl.ds(0, EB)], ssem).wait()

    def grp_prologue(gi):
        pltpu.sync_copy(idx_hbm.at[pl.ds(tb + gi * GRP, GRP)], idxv)
        for b in range(GRP):
            for k in range(EB // 16):
                idxv[b, 0, pl.ds(k * 16, 16)] = (
                    idxv[b, 0, pl.ds(k * 16, 16)] + coff)
        g_start(0, 0)
        g_start(1, 1)

    def grp_main():
        for b in range(GRP):
            sl = b % NSL
            g_wait()
            s_start(b, sl)
            if b + 2 < GRP:
                if b >= 1:
                    s_wait()
                g_start(b + 2, (b + 2) % NSL)

    def grp(gi, carry):
        grp_main()
        pltpu.sync_copy(idx_hbm.at[pl.ds(tb + gi * GRP, GRP)], idxv)
        for b in range(GRP):
            for k in range(EB // 16):
                idxv[b, 0, pl.ds(k * 16, 16)] = (
                    idxv[b, 0, pl.ds(k * 16, 16)] + coff)
        s_wait()
        s_wait()
        s_wait()
        g_start(0, 0)
        g_start(1, 1)
        return carry

    grp_prologue(0)

    @pl.when(s < NS - 1)
    def _():
        pltpu.sync_copy(bias_hbm.at[c], acc.at[pl.ds(base, STRIPE)])

    @pl.when(s == NS - 1)
    def _():
        pltpu.sync_copy(bias_hbm.at[c, pl.ds(0, LAST_INIT)],
                        acc.at[pl.ds(base, LAST_INIT)])

    plsc.subcore_barrier()

    lax.fori_loop(1, N_GRPS, grp, None)
    grp_main()
    s_wait()
    s_wait()
    s_wait()

    plsc.subcore_barrier()

    @pl.when(s < NS - 1)
    def _():
        pltpu.sync_copy(acc.at[pl.ds(base, STRIPE)],
                        out_hbm.at[pl.ds(base, STRIPE), pl.ds(c * H, H)])

    @pl.when(s == NS - 1)
    def _():
        pltpu.sync_copy(acc.at[pl.ds(base, LAST_OUT)],
                        out_hbm.at[pl.ds(base, LAST_OUT), pl.ds(c * H, H)])


_sc_scatter = functools.partial(
    pl.kernel,
    out_type=jax.ShapeDtypeStruct((N_NODES, D), jnp.float32),
    mesh=plsc.VectorSubcoreMesh(core_axis_name="c", subcore_axis_name="s",
                                num_cores=NC, num_subcores=NS),
    scratch_types=[
        pltpu.VMEM_SHARED((ACC_ROWS, H), jnp.float32),
        pltpu.VMEM((GRP, 2, EB), jnp.int32),
        pltpu.VMEM((NSL, EB), jnp.int32),
        pltpu.VMEM((NSL * EB, H), jnp.float32),
        pltpu.SemaphoreType.DMA,
        pltpu.SemaphoreType.DMA,
    ],
)(_sc_body)


def kernel(inputs, edge_index, weight, bias):
    row = edge_index[0].astype(jnp.int32)
    col = edge_index[1].astype(jnp.int32)

    rowp = jnp.full((E_PAD,), DUMMY_ROW, jnp.int32).at[:N_EDGES].set(row)
    colp = jnp.zeros((E_PAD,), jnp.int32).at[:N_EDGES].set(col)
    idx = jnp.stack([colp.reshape(TOTAL_BLOCKS, EB),
                     rowp.reshape(TOTAL_BLOCKS, EB)], axis=1)

    t2 = _matmul_halves(inputs, weight)
    biasfill = jnp.broadcast_to(bias.reshape(NC, 1, H), (NC, STRIPE, H))
    return _sc_scatter(t2, idx, biasfill)
